# baseline (speedup 1.0000x reference)
import functools
import jax
import jax.numpy as jnp
from jax import lax
from jax.experimental import pallas as pl
from jax.experimental.pallas import tpu as pltpu
from jax.experimental.pallas import tpu_sc as plsc

_N = 96
_P = _N * _N
_XPW = 3


def _bitcast_i(x):
    return plsc.bitcast(x, jnp.int32)


def _bitcast_f(x):
    return plsc.bitcast(x, jnp.float32)


def _build_e(u6, qz, e2, sx, pa, eslot):
    def row(y, c):
        rb_a = pa * _P + y * _N
        rb_b = rb_a + _P
        for j in range(6):
            s = u6[pl.ds(rb_a + j * 16, 16)] + u6[pl.ds(rb_b + j * 16, 16)]
            sx[pl.ds(j * 16, 16)] = s
            if j == 0:
                sx[pl.ds(_N, 16)] = s
        for j in range(6):
            qz[pl.ds(y * _N + j * 16, 16)] = (
                sx[pl.ds(j * 16, 16)] + sx[pl.ds(j * 16 + 1, 16)]
            )
        return c

    lax.fori_loop(0, _N, row, 0)

    def row2(y, c):
        yb = y * _N
        y1b = lax.rem(y + 1, _N) * _N
        for j in range(6):
            e2[pl.ds(eslot * _P + yb + j * 16, 16)] = (
                qz[pl.ds(yb + j * 16, 16)] + qz[pl.ds(y1b + j * 16, 16)]
            )
        return c

    lax.fori_loop(0, _N, row2, 0)


def _combine(u6, t3, e2, v1, sx, na_v, dpa_v, q, s0, s1):
    neg_a = na_v[pl.ds(0, 16)]
    d_plus_a = dpa_v[pl.ds(0, 16)]

    def row(y, c):
        ym1 = lax.rem(y + _N - 1, _N)
        rbs = []
        for si, slot in enumerate((s0, s1)):
            for yi, yy in enumerate((ym1, y)):
                rb = slot * _P + yy * _N
                k = 32 * (si * 2 + yi)
                sx[pl.ds(k, 16)] = e2[pl.ds(rb + 5 * 16, 16)]
                sx[pl.ds(k + 16, 16)] = e2[pl.ds(rb, 16)]
                rbs.append((rb, k))
        for j in range(6):
            zb = j * 16
            ez = []
            for rb, k in rbs:
                if j > 0:
                    em = e2[pl.ds(rb + zb - 1, 16)]
                else:
                    em = sx[pl.ds(k + 15, 16)]
                e0 = e2[pl.ds(rb + zb, 16)]
                ez.append((em, e0))
            t = t3[pl.ds(q * _P + y * _N + zb, 16)]
            zero = jnp.zeros((16,), jnp.float32)
            one = zero + 1.0
            pc = zero
            acc = zero
            for p0 in (0, 1):
                for p1 in (0, 1):
                    pair = ez[p0 * 2 + p1]
                    for p2 in (0, 1):
                        e = p0 * 4 + p1 * 2 + p2
                        m = (t << (31 - e)) < 0
                        pc = pc + jnp.where(m, one, zero)
                        acc = acc + jnp.where(m, pair[p2], zero)
            u = u6[pl.ds((q + 1) * _P + y * _N + zb, 16)]
            v1[pl.ds(y * _N + zb, 16)] = d_plus_a * (u * pc) + neg_a * acc
        return c

    lax.fori_loop(0, _N, row, 0)


def _sc_call(u_flat, t_flat, na16, dpa16):
    mesh = plsc.VectorSubcoreMesh(core_axis_name="c", subcore_axis_name="s")

    @functools.partial(
        pl.kernel,
        mesh=mesh,
        out_type=jax.ShapeDtypeStruct((_N * _N * _N,), jnp.float32),
        scratch_types=[
            pltpu.MemorySpace.VMEM((6 * _P,), jnp.float32),
            pltpu.MemorySpace.VMEM((_XPW * _P,), jnp.int32),
            pltpu.MemorySpace.VMEM((2 * _P,), jnp.float32),
            pltpu.MemorySpace.VMEM((_P,), jnp.float32),
            pltpu.MemorySpace.VMEM((_P,), jnp.float32),
            pltpu.MemorySpace.VMEM((128,), jnp.float32),
            pltpu.MemorySpace.VMEM((16,), jnp.float32),
            pltpu.MemorySpace.VMEM((16,), jnp.float32),
        ],
    )
    def sck(
        u_hbm, t_hbm, na_hbm, dpa_hbm, out_hbm,
        u6, t3, e2, qz, v1, sx, na_v, dpa_v,
    ):
        w = lax.axis_index("s") * 2 + lax.axis_index("c")
        x0 = w * _XPW
        pltpu.sync_copy(
            u_hbm.at[pl.ds(lax.rem(x0 + _N - 1, _N) * _P, _P)],
            u6.at[pl.ds(0, _P)],
        )
        pltpu.sync_copy(
            u_hbm.at[pl.ds(x0 * _P, 3 * _P)], u6.at[pl.ds(_P, 3 * _P)]
        )
        pltpu.sync_copy(
            u_hbm.at[pl.ds(lax.rem(x0 + 3, _N) * _P, 2 * _P)],
            u6.at[pl.ds(4 * _P, 2 * _P)],
        )
        pltpu.sync_copy(t_hbm.at[pl.ds(x0 * _P, _XPW * _P)], t3)
        pltpu.sync_copy(na_hbm, na_v)
        pltpu.sync_copy(dpa_hbm, dpa_v)

        _build_e(u6, qz, e2, sx, 0, 0)
        for q in range(_XPW):
            _build_e(u6, qz, e2, sx, q + 1, (q + 1) % 2)
            _combine(u6, t3, e2, v1, sx, na_v, dpa_v, q, q % 2, (q + 1) % 2)
            pltpu.sync_copy(v1, out_hbm.at[pl.ds((x0 + q) * _P, _P)])

    return sck(u_flat, t_flat, na16, dpa16)


def kernel(U, H8types, filters):
    na16 = jnp.full((16,), filters[1, 0], jnp.float32)
    dpa16 = jnp.full((16,), filters[1, 13] - filters[1, 0], jnp.float32)
    out = _sc_call(U.reshape(-1), H8types.reshape(-1), na16, dpa16)
    return out.reshape(U.shape)


# speedup vs baseline: 6.8026x; 6.8026x over previous
import jax
import jax.numpy as jnp
from jax import lax
from jax.experimental import pallas as pl
from jax.experimental.pallas import tpu as pltpu

_N = 96
_BX = 16
_G = _N // _BX


def _slab_copy(u_hbm, u_vmem, sems, j):
    return pltpu.make_async_copy(
        u_hbm.at[pl.ds(j * _BX, _BX)],
        u_vmem.at[pl.ds(j * _BX, _BX)],
        sems.at[j],
    )


def _body(u_hbm, t_ref, f_ref, out_ref, u_vmem, sems):
    i = pl.program_id(0)

    @pl.when(i == 0)
    def _():
        _slab_copy(u_hbm, u_vmem, sems, _G - 1).start()
        _slab_copy(u_hbm, u_vmem, sems, 0).start()
        _slab_copy(u_hbm, u_vmem, sems, 1).start()
        _slab_copy(u_hbm, u_vmem, sems, 2).start()
        _slab_copy(u_hbm, u_vmem, sems, _G - 1).wait()
        _slab_copy(u_hbm, u_vmem, sems, 0).wait()
        _slab_copy(u_hbm, u_vmem, sems, 1).wait()

    @pl.when((i > 0) & (i < _G - 3))
    def _():
        pltpu.make_async_copy(
            u_hbm.at[pl.ds((i + 2) * _BX, _BX)],
            u_vmem.at[pl.ds((i + 2) * _BX, _BX)],
            sems.at[i + 2],
        ).start()

    @pl.when((i > 0) & (i < _G - 2))
    def _():
        pltpu.make_async_copy(
            u_hbm.at[pl.ds((i + 1) * _BX, _BX)],
            u_vmem.at[pl.ds((i + 1) * _BX, _BX)],
            sems.at[i + 1],
        ).wait()

    neg_a = f_ref[1, 0]
    d_plus_a = f_ref[1, 13] - f_ref[1, 0]

    x0 = i * _BX
    lo = (x0 + (_N - 1)) % _N
    hi = (x0 + _BX) % _N
    t = t_ref[...]

    Ex = jnp.concatenate(
        [
            u_vmem[pl.ds(lo, 1)] + u_vmem[pl.ds(x0, 1)],
            u_vmem[pl.ds(x0, _BX - 1)] + u_vmem[pl.ds(x0 + 1, _BX - 1)],
            u_vmem[pl.ds(x0 + _BX - 1, 1)] + u_vmem[pl.ds(hi, 1)],
            u_vmem[pl.ds(hi, 1)] + u_vmem[pl.ds(hi + 1, 1)],
        ],
        axis=0,
    )
    Exy = Ex + jnp.roll(Ex, -1, 1)
    E = Exy + jnp.roll(Exy, -1, 2)

    e_yz = {
        (1, 1): E,
        (1, 0): jnp.roll(E, 1, 2),
        (0, 1): jnp.roll(E, 1, 1),
    }
    e_yz[(0, 0)] = jnp.roll(e_yz[(1, 0)], 1, 1)

    acc_i = jnp.zeros((_BX, _N, _N), jnp.int32)
    acc = jnp.zeros((_BX, _N, _N), jnp.float32)
    for p1 in (0, 1):
        for p2 in (0, 1):
            eyz = e_yz[(p1, p2)]
            eyz_x0 = lax.bitcast_convert_type(eyz[1 : _BX + 1], jnp.int32)
            eyz_xm1 = lax.bitcast_convert_type(eyz[:_BX], jnp.int32)
            for p0 in (0, 1):
                e = p0 * 4 + p1 * 2 + p2
                m = (t << (31 - e)) >> 31
                acc_i = acc_i + m
                acc = acc + lax.bitcast_convert_type(
                    m & (eyz_x0 if p0 else eyz_xm1), jnp.float32
                )
    U0 = u_vmem[pl.ds(x0, _BX)]
    pc = (-acc_i).astype(jnp.float32)
    out_ref[...] = d_plus_a * (U0 * pc) + neg_a * acc


def kernel(U, H8types, filters):
    return pl.pallas_call(
        _body,
        grid=(_G,),
        in_specs=[
            pl.BlockSpec(memory_space=pltpu.MemorySpace.HBM),
            pl.BlockSpec((_BX, _N, _N), lambda i: (i, 0, 0)),
            pl.BlockSpec((256, 27), lambda i: (0, 0)),
        ],
        out_specs=pl.BlockSpec((_BX, _N, _N), lambda i: (i, 0, 0)),
        out_shape=jax.ShapeDtypeStruct(U.shape, U.dtype),
        scratch_shapes=[
            pltpu.MemorySpace.VMEM((_N, _N, _N), jnp.float32),
            pltpu.SemaphoreType.DMA((_G,)),
        ],
    )(U, H8types, filters)
